# baseline (speedup 1.0000x reference)
import functools

import jax
import jax.numpy as jnp
from jax import lax
from jax.experimental import pallas as pl
from jax.experimental.pallas import tpu as pltpu
from jax.experimental.pallas import tpu_sc as plsc

_K = 20
_KPAD = 32
_NW = 32
_L = 16
_INF = float("inf")


def _rev(x):
    return lax.rev(x, dimensions=(0,))


def _sort_pair(k, v):
    return plsc.sort_key_val(k, v)


def _merge32(lo0, loi0, hi0, hii0, sk, si):
    rk, ri = _rev(sk), _rev(si)
    m = hi0 <= rk
    a = jnp.where(m, hi0, rk)
    ai = jnp.where(m, hii0, ri)
    a, ai = _sort_pair(a, ai)
    rk2, ri2 = _rev(a), _rev(ai)
    m2 = lo0 <= rk2
    lo = jnp.where(m2, lo0, rk2)
    loi = jnp.where(m2, loi0, ri2)
    hi = jnp.where(m2, rk2, lo0)
    hii = jnp.where(m2, ri2, loi0)
    lo, loi = _sort_pair(lo, loi)
    hi, hii = _sort_pair(hi, hii)
    return lo, loi, hi, hii


def _make_sc_knn(F):
    QPW = F // _NW
    NCHUNK = F // _L
    mesh = plsc.VectorSubcoreMesh(core_axis_name="c", subcore_axis_name="s",
                                  num_cores=2, num_subcores=16)

    @functools.partial(
        pl.kernel,
        mesh=mesh,
        compiler_params=pltpu.CompilerParams(needs_layout_passes=False),
        out_type=jax.ShapeDtypeStruct((F, _KPAD), jnp.int32),
        scratch_types=[
            pltpu.VMEM((F + _L,), jnp.float32),
            pltpu.VMEM((F + _L,), jnp.float32),
            pltpu.VMEM((F + _L,), jnp.float32),
            pltpu.VMEM((F,), jnp.float32),
            pltpu.VMEM((F + _L,), jnp.int32),
            pltpu.VMEM((QPW, _KPAD), jnp.int32),
            pltpu.SemaphoreType.DMA,
        ],
    )
    def knn(cx_hbm, cy_hbm, cz_hbm, out_hbm, cx, cy, cz, dbuf, cand_i, outbuf, sem):
        wid = lax.axis_index("s") * 2 + lax.axis_index("c")
        base = wid * QPW
        pltpu.sync_copy(cx_hbm, cx.at[pl.ds(0, F)])
        pltpu.sync_copy(cy_hbm, cy.at[pl.ds(0, F)])
        pltpu.sync_copy(cz_hbm, cz.at[pl.ds(0, F)])
        iota = lax.iota(jnp.int32, _L)

        def per_query(q, carry):
            qxv = cx[pl.ds(base + q, _L)]
            qyv = cy[pl.ds(base + q, _L)]
            qzv = cz[pl.ds(base + q, _L)]
            qx = jnp.full((_L,), qxv[0], jnp.float32)
            qy = jnp.full((_L,), qyv[0], jnp.float32)
            qz = jnp.full((_L,), qzv[0], jnp.float32)

            def p1(i, accs):
                acc_a, acc_b = accs
                o0 = i * (4 * _L)
                for u in range(4):
                    o = o0 + u * _L
                    dx = cx[pl.ds(o, _L)] - qx
                    dy = cy[pl.ds(o, _L)] - qy
                    dz = cz[pl.ds(o, _L)] - qz
                    d2 = (dx * dx + dy * dy) + dz * dz
                    dbuf[pl.ds(o, _L)] = d2
                    if u < 2:
                        acc_a = jnp.minimum(acc_a, d2)
                    else:
                        acc_b = jnp.minimum(acc_b, d2)
                return (acc_a, acc_b)

            acc_a, acc_b = lax.fori_loop(
                0, NCHUNK // 4, p1,
                (jnp.full((_L,), _INF), jnp.full((_L,), _INF)))

            s_a, _ = _sort_pair(acc_a, acc_a)
            s_b, _ = _sort_pair(acc_b, acc_b)
            up = jnp.maximum(s_a, _rev(s_b))
            s_u, _ = _sort_pair(up, up)
            thr = jnp.full((_L,), s_u[_K - 1 - _L], jnp.float32)

            def p2(i, cnt):
                o0 = i * (2 * _L)
                v1 = dbuf[pl.ds(o0, _L)]
                v2 = dbuf[pl.ds(o0 + _L, _L)]
                m1 = v1 <= thr
                m2 = v2 <= thr

                def hit(c):
                    plsc.store_compressed(cand_i.at[pl.ds(c, _L)], iota + o0,
                                          mask=m1)
                    c1 = c + plsc.all_reduce_population_count(m1)[0]
                    plsc.store_compressed(cand_i.at[pl.ds(c1, _L)],
                                          iota + (o0 + _L), mask=m2)
                    return c1 + plsc.all_reduce_population_count(m2)[0]

                return lax.cond(jnp.any(m1 | m2), hit, lambda c: c, cnt)

            cnt = lax.fori_loop(0, NCHUNK // 2, p2, jnp.int32(0))

            def p3(j, state):
                lo, loi, hi, hii = state
                iv = cand_i[pl.ds(j * _L, _L)]
                valid = (iota + j * _L) < cnt
                iv = jnp.where(valid, iv, 0)
                vd = plsc.load_gather(dbuf, [iv])
                key = jnp.where(valid, vd, _INF)
                sk, si = _sort_pair(key, iv)
                return _merge32(lo, loi, hi, hii, sk, si)

            nch = (cnt + (_L - 1)) // _L
            init = (jnp.full((_L,), _INF), jnp.zeros((_L,), jnp.int32),
                    jnp.full((_L,), _INF), jnp.zeros((_L,), jnp.int32))
            lo, loi, hi, hii = lax.fori_loop(0, nch, p3, init)

            outbuf[q, pl.ds(0, _L)] = loi
            outbuf[q, pl.ds(_L, _L)] = hii
            return carry

        lax.fori_loop(0, QPW, per_query, jnp.int32(0))
        pltpu.sync_copy(outbuf, out_hbm.at[pl.ds(base, QPW)])

    return knn


def _crossing_body(nex_ref, ney_ref, nez_ref, ex_ref, ey_ref, ez_ref,
                   eax_ref, eay_ref, eaz_ref, probs_ref, out_ref):
    nex = nex_ref[...]
    ney = ney_ref[...]
    nez = nez_ref[...]
    eax = eax_ref[...]
    eay = eay_ref[...]
    eaz = eaz_ref[...]
    acc = jnp.zeros_like(nex)
    for j in range(3):
        e_x = ex_ref[:, j:j + 1]
        e_y = ey_ref[:, j:j + 1]
        e_z = ez_ref[:, j:j + 1]
        cp_x = e_y * nez - e_z * ney
        cp_y = e_z * nex - e_x * nez
        cp_z = e_x * ney - e_y * nex
        denom = (cp_x * e_x + cp_y * e_y) + cp_z * e_z
        tnum = (cp_x * nex + cp_y * ney) + cp_z * nez
        unum = (cp_x * eax + cp_y * eay) + cp_z * eaz
        t = tnum / denom
        u = unum / denom
        mask = (t >= 0.0) & (t <= 1.0) & (u >= 0.0) & (u <= 1.0)
        acc = acc + jnp.where(mask, 1.0, 0.0)
    weighted = acc * probs_ref[...]
    out_ref[...] = jnp.sum(weighted, axis=(0, 1), keepdims=True)


def kernel(vertices, faces, face_probs):
    F = faces.shape[0]
    k = min(_K, F)
    centroids = vertices[faces].mean(axis=1)

    knn = _make_sc_knn(F)
    nearest = knn(centroids[:, 0].copy(), centroids[:, 1].copy(),
                  centroids[:, 2].copy())[:, :k]

    perm = jnp.array([1, 2, 0])
    edges = vertices[faces[:, perm]] - vertices[faces]
    nf = faces[nearest]
    ne = vertices[nf[:, :, perm]] - vertices[nf]

    nex = ne[..., 0].reshape(F, k * 3)
    ney = ne[..., 1].reshape(F, k * 3)
    nez = ne[..., 2].reshape(F, k * 3)
    ex = edges[..., 0]
    ey = edges[..., 1]
    ez = edges[..., 2]
    eax = jnp.tile(ex, (1, k))
    eay = jnp.tile(ey, (1, k))
    eaz = jnp.tile(ez, (1, k))
    probs2 = face_probs[:F, None]

    out = pl.pallas_call(
        _crossing_body,
        out_shape=jax.ShapeDtypeStruct((1, 1), jnp.float32),
    )(nex, ney, nez, ex, ey, ez, eax, eay, eaz, probs2)
    return out[0, 0]


# speedup vs baseline: 4.8811x; 4.8811x over previous
import functools

import jax
import jax.numpy as jnp
from jax import lax
from jax.experimental import pallas as pl
from jax.experimental.pallas import tpu as pltpu
from jax.experimental.pallas import tpu_sc as plsc

_K = 20
_NW = 32
_L = 16
_INF = float("inf")


def _rev(x):
    return lax.rev(x, dimensions=(0,))


def _sort_pair(k, v):
    return plsc.sort_key_val(k, v)


def _merge32(lo0, loi0, hi0, hii0, sk, si):
    rk, ri = _rev(sk), _rev(si)
    m = hi0 <= rk
    a = jnp.where(m, hi0, rk)
    ai = jnp.where(m, hii0, ri)
    a, ai = _sort_pair(a, ai)
    rk2, ri2 = _rev(a), _rev(ai)
    m2 = lo0 <= rk2
    lo = jnp.where(m2, lo0, rk2)
    loi = jnp.where(m2, loi0, ri2)
    hi = jnp.where(m2, rk2, lo0)
    hii = jnp.where(m2, ri2, loi0)
    lo, loi = _sort_pair(lo, loi)
    hi, hii = _sort_pair(hi, hii)
    return lo, loi, hi, hii


def _make_sc_knn_gather(F, V):
    QPW = F // _NW
    NCHUNK = F // _L
    W = 3 * _K
    mesh = plsc.VectorSubcoreMesh(core_axis_name="c", subcore_axis_name="s",
                                  num_cores=2, num_subcores=16)

    @functools.partial(
        pl.kernel,
        mesh=mesh,
        compiler_params=pltpu.CompilerParams(needs_layout_passes=False),
        out_type=(jax.ShapeDtypeStruct((F, W), jnp.float32),
                  jax.ShapeDtypeStruct((F, W), jnp.float32),
                  jax.ShapeDtypeStruct((F, W), jnp.float32)),
        scratch_types=[
            pltpu.VMEM((F + _L,), jnp.float32),
            pltpu.VMEM((F + _L,), jnp.float32),
            pltpu.VMEM((F + _L,), jnp.float32),
            pltpu.VMEM((F,), jnp.float32),
            pltpu.VMEM((F + _L,), jnp.int32),
            pltpu.VMEM((V,), jnp.float32),
            pltpu.VMEM((V,), jnp.float32),
            pltpu.VMEM((V,), jnp.float32),
            pltpu.VMEM((F,), jnp.int32),
            pltpu.VMEM((F,), jnp.int32),
            pltpu.VMEM((F,), jnp.int32),
            pltpu.VMEM((QPW, W), jnp.float32),
            pltpu.VMEM((QPW, W), jnp.float32),
            pltpu.VMEM((QPW, W), jnp.float32),
            pltpu.SemaphoreType.DMA,
        ],
    )
    def knn(cx_hbm, cy_hbm, cz_hbm, vx_hbm, vy_hbm, vz_hbm,
            f0_hbm, f1_hbm, f2_hbm,
            nex_hbm, ney_hbm, nez_hbm,
            cx, cy, cz, dbuf, cand_i, vx, vy, vz, f0, f1, f2,
            nxb, nyb, nzb, sem):
        wid = lax.axis_index("s") * 2 + lax.axis_index("c")
        base = wid * QPW
        pltpu.sync_copy(cx_hbm, cx.at[pl.ds(0, F)])
        pltpu.sync_copy(cy_hbm, cy.at[pl.ds(0, F)])
        pltpu.sync_copy(cz_hbm, cz.at[pl.ds(0, F)])
        pltpu.sync_copy(vx_hbm, vx)
        pltpu.sync_copy(vy_hbm, vy)
        pltpu.sync_copy(vz_hbm, vz)
        pltpu.sync_copy(f0_hbm, f0)
        pltpu.sync_copy(f1_hbm, f1)
        pltpu.sync_copy(f2_hbm, f2)
        iota = lax.iota(jnp.int32, _L)

        def per_query(q, carry):
            qxv = cx[pl.ds(base + q, _L)]
            qyv = cy[pl.ds(base + q, _L)]
            qzv = cz[pl.ds(base + q, _L)]
            qx = jnp.full((_L,), qxv[0], jnp.float32)
            qy = jnp.full((_L,), qyv[0], jnp.float32)
            qz = jnp.full((_L,), qzv[0], jnp.float32)

            def p1(i, accs):
                acc_a, acc_b = accs
                o0 = i * (4 * _L)
                for u in range(4):
                    o = o0 + u * _L
                    dx = cx[pl.ds(o, _L)] - qx
                    dy = cy[pl.ds(o, _L)] - qy
                    dz = cz[pl.ds(o, _L)] - qz
                    d2 = (dx * dx + dy * dy) + dz * dz
                    dbuf[pl.ds(o, _L)] = d2
                    if u < 2:
                        acc_a = jnp.minimum(acc_a, d2)
                    else:
                        acc_b = jnp.minimum(acc_b, d2)
                return (acc_a, acc_b)

            acc_a, acc_b = lax.fori_loop(
                0, NCHUNK // 4, p1,
                (jnp.full((_L,), _INF), jnp.full((_L,), _INF)))

            s_a, _ = _sort_pair(acc_a, acc_a)
            s_b, _ = _sort_pair(acc_b, acc_b)
            up = jnp.maximum(s_a, _rev(s_b))
            s_u, _ = _sort_pair(up, up)
            thr = jnp.full((_L,), s_u[_K - 1 - _L], jnp.float32)

            def p2(i, cnt):
                o0 = i * (2 * _L)
                v1 = dbuf[pl.ds(o0, _L)]
                v2 = dbuf[pl.ds(o0 + _L, _L)]
                m1 = v1 <= thr
                m2 = v2 <= thr

                def hit(c):
                    plsc.store_compressed(cand_i.at[pl.ds(c, _L)], iota + o0,
                                          mask=m1)
                    c1 = c + plsc.all_reduce_population_count(m1)[0]
                    plsc.store_compressed(cand_i.at[pl.ds(c1, _L)],
                                          iota + (o0 + _L), mask=m2)
                    return c1 + plsc.all_reduce_population_count(m2)[0]

                return lax.cond(jnp.any(m1 | m2), hit, lambda c: c, cnt)

            cnt = lax.fori_loop(0, NCHUNK // 2, p2, jnp.int32(0))

            def p3(j, state):
                lo, loi, hi, hii = state
                iv = cand_i[pl.ds(j * _L, _L)]
                valid = (iota + j * _L) < cnt
                iv = jnp.where(valid, iv, 0)
                vd = plsc.load_gather(dbuf, [iv])
                key = jnp.where(valid, vd, _INF)
                sk, si = _sort_pair(key, iv)
                return _merge32(lo, loi, hi, hii, sk, si)

            nch = (cnt + (_L - 1)) // _L
            init = (jnp.full((_L,), _INF), jnp.zeros((_L,), jnp.int32),
                    jnp.full((_L,), _INF), jnp.zeros((_L,), jnp.int32))
            lo, loi, hi, hii = lax.fori_loop(0, nch, p3, init)

            qfull = jnp.full((_L,), q, jnp.int32)
            for chunk, (ids_raw, msk) in enumerate(
                    ((loi, None), (hii, iota < 4))):
                ids = ids_raw if msk is None else jnp.where(msk, ids_raw, 0)
                fcols = (plsc.load_gather(f0, [ids]),
                         plsc.load_gather(f1, [ids]),
                         plsc.load_gather(f2, [ids]))
                vcoords = []
                for fcol in fcols:
                    vcoords.append((plsc.load_gather(vx, [fcol]),
                                    plsc.load_gather(vy, [fcol]),
                                    plsc.load_gather(vz, [fcol])))
                colbase = (iota + chunk * _L) * 3
                for a in range(3):
                    nxt = vcoords[(a + 1) % 3]
                    cur = vcoords[a]
                    col = colbase + a
                    for comp, obuf in ((0, nxb), (1, nyb), (2, nzb)):
                        val = nxt[comp] - cur[comp]
                        plsc.store_scatter(obuf, [qfull, col], val, mask=msk)
            return carry

        lax.fori_loop(0, QPW, per_query, jnp.int32(0))
        pltpu.sync_copy(nxb, nex_hbm.at[pl.ds(base, QPW)])
        pltpu.sync_copy(nyb, ney_hbm.at[pl.ds(base, QPW)])
        pltpu.sync_copy(nzb, nez_hbm.at[pl.ds(base, QPW)])

    return knn


def _crossing_body(nex_ref, ney_ref, nez_ref, ex_ref, ey_ref, ez_ref,
                   eax_ref, eay_ref, eaz_ref, probs_ref, out_ref):
    nex = nex_ref[...]
    ney = ney_ref[...]
    nez = nez_ref[...]
    eax = eax_ref[...]
    eay = eay_ref[...]
    eaz = eaz_ref[...]
    acc = jnp.zeros_like(nex)
    for j in range(3):
        e_x = ex_ref[:, j:j + 1]
        e_y = ey_ref[:, j:j + 1]
        e_z = ez_ref[:, j:j + 1]
        cp_x = e_y * nez - e_z * ney
        cp_y = e_z * nex - e_x * nez
        cp_z = e_x * ney - e_y * nex
        denom = (cp_x * e_x + cp_y * e_y) + cp_z * e_z
        tnum = (cp_x * nex + cp_y * ney) + cp_z * nez
        unum = (cp_x * eax + cp_y * eay) + cp_z * eaz
        t = tnum / denom
        u = unum / denom
        mask = (t >= 0.0) & (t <= 1.0) & (u >= 0.0) & (u <= 1.0)
        acc = acc + jnp.where(mask, 1.0, 0.0)
    weighted = acc * probs_ref[...]
    out_ref[...] = jnp.sum(weighted, axis=(0, 1), keepdims=True)


def kernel(vertices, faces, face_probs):
    F = faces.shape[0]
    V = vertices.shape[0]
    k = min(_K, F)
    centroids = vertices[faces].mean(axis=1)

    knn = _make_sc_knn_gather(F, V)
    nex, ney, nez = knn(
        centroids[:, 0].copy(), centroids[:, 1].copy(), centroids[:, 2].copy(),
        vertices[:, 0].copy(), vertices[:, 1].copy(), vertices[:, 2].copy(),
        faces[:, 0].copy(), faces[:, 1].copy(), faces[:, 2].copy())

    perm = jnp.array([1, 2, 0])
    edges = vertices[faces[:, perm]] - vertices[faces]
    ex = edges[..., 0]
    ey = edges[..., 1]
    ez = edges[..., 2]
    eax = jnp.tile(ex, (1, k))
    eay = jnp.tile(ey, (1, k))
    eaz = jnp.tile(ez, (1, k))
    probs2 = face_probs[:F, None]

    out = pl.pallas_call(
        _crossing_body,
        out_shape=jax.ShapeDtypeStruct((1, 1), jnp.float32),
    )(nex, ney, nez, ex, ey, ez, eax, eay, eaz, probs2)
    return out[0, 0]
